# baseline (speedup 1.0000x reference)
import functools

import jax
import jax.numpy as jnp
from jax import lax
from jax.experimental import pallas as pl
from jax.experimental.pallas import tpu as pltpu
from jax.experimental.pallas import tpu_sc as plsc

N = 1_000_000
D = 64
K = 2048
NPAD = 1 << 20
BLK = 8192
GRID = NPAD // BLK
_MINT = -2147483648

NC = 2
NS = 16
NW = NC * NS
PER_TILE = NPAD // NW
ROWS_PER_TILE = K // NW


def _score_body(x_ref, p_ref, o_ref):
    i = pl.program_id(0)
    pv = p_ref[...]
    nrm = jnp.sqrt(jnp.sum(pv * pv))
    y2 = lax.dot_general(x_ref[...], pv.reshape(D, 1), (((1,), (0,)), ((), ())),
                         preferred_element_type=jnp.float32)
    y = (y2 / nrm).reshape(BLK)
    s = lax.bitcast_convert_type(y, jnp.int32)
    ou = jnp.where(s < 0, ~s, s | jnp.int32(_MINT))
    gid = i * BLK + lax.broadcasted_iota(jnp.int32, (BLK,), 0)
    o_ref[...] = lax.bitcast_convert_type(
        jnp.where(gid < N, ou, jnp.int32(0)), jnp.uint32)


def _scores(x, p):
    return pl.pallas_call(
        _score_body,
        out_shape=jax.ShapeDtypeStruct((NPAD,), jnp.uint32),
        grid=(GRID,),
        in_specs=[
            pl.BlockSpec((BLK, D), lambda i: (jnp.minimum(i, N // BLK), 0)),
            pl.BlockSpec((1, D), lambda i: (0, 0)),
        ],
        out_specs=pl.BlockSpec((BLK,), lambda i: (i,)),
    )(x, p.reshape(1, D))


def _rank_body(or_ref, oc_ref, ir_ref, ic_ref, si_ref, sm_ref):
    oi_row = lax.bitcast_convert_type(or_ref[...], jnp.int32) ^ jnp.int32(_MINT)
    idx_row = ir_ref[...]
    ranks = jnp.zeros((1, K), jnp.int32)
    for c in range(K // 128):
        ocb = lax.bitcast_convert_type(
            oc_ref[pl.ds(c * 128, 128), :], jnp.int32) ^ jnp.int32(_MINT)
        icb = ic_ref[pl.ds(c * 128, 128), :]
        beats = (ocb > oi_row) | ((ocb == oi_row) & (icb < idx_row))
        ranks = ranks + jnp.sum(beats.astype(jnp.int32), axis=0, keepdims=True)
    for c in range(K // 128):
        rcol = c * 128 + lax.broadcasted_iota(jnp.int32, (128, 1), 0)
        onehot = ranks == rcol
        si_ref[pl.ds(c * 128, 128), :] = jnp.sum(
            jnp.where(onehot, idx_row, 0), axis=1, keepdims=True)
        soi = jnp.sum(jnp.where(onehot, oi_row, 0), axis=1, keepdims=True)
        s = soi ^ ((soi >> 31) & jnp.int32(0x7FFFFFFF))
        val = lax.bitcast_convert_type(s, jnp.float32)
        sm_ref[pl.ds(c * 128, 128), :] = jnp.broadcast_to(
            jnp.tanh(val), (128, D))


def _rank(win_ord, win_idx):
    return pl.pallas_call(
        _rank_body,
        out_shape=(
            jax.ShapeDtypeStruct((K, 1), jnp.int32),
            jax.ShapeDtypeStruct((K, D), jnp.float32),
        ),
    )(
        win_ord.reshape(1, K),
        win_ord.reshape(K, 1),
        win_idx.reshape(1, K),
        win_idx.reshape(K, 1),
    )


def _gather_body(x2_hbm, si_hbm, sm2_hbm, out2_hbm, idx_all_v, idx_v, idx2_v,
                 rows2_v, buf2_v, scl2_v, sem):
    wid = lax.axis_index("s") * NC + lax.axis_index("c")
    base = wid * ROWS_PER_TILE
    base2 = pl.multiple_of(wid * (ROWS_PER_TILE // 2), ROWS_PER_TILE // 2)
    pltpu.sync_copy(si_hbm, idx_all_v)
    for c in range(ROWS_PER_TILE // 16):
        sl = pl.ds(c * 16, 16)
        chunk = idx_all_v[pl.ds(base + c * 16, 16)]
        idx_v[sl] = chunk
        idx2_v[sl] = lax.shift_right_logical(chunk, 1)
    pltpu.async_copy(x2_hbm.at[idx2_v], rows2_v, sem).wait()
    pltpu.sync_copy(sm2_hbm.at[pl.ds(base2, ROWS_PER_TILE // 2)], scl2_v)
    for r in range(ROWS_PER_TILE):
        odd = (idx_v[pl.ds((r // 16) * 16, 16)][r % 16] & 1) == 1
        dst_r = r // 2
        dst_c = (r % 2) * D
        for c in range(D // 16):
            lo = rows2_v[r, pl.ds(c * 16, 16)]
            hi = rows2_v[r, pl.ds(D + c * 16, 16)]
            dst = pl.ds(dst_c + c * 16, 16)
            buf2_v[dst_r, dst] = jnp.where(odd, hi, lo) * scl2_v[dst_r, dst]
    pltpu.sync_copy(buf2_v, out2_hbm.at[pl.ds(base2, ROWS_PER_TILE // 2)])


def _gather_scale(x, sorted_idx, scale_mat):
    mesh = plsc.VectorSubcoreMesh(core_axis_name="c", subcore_axis_name="s")
    fn = functools.partial(
        pl.kernel,
        mesh=mesh,
        out_type=jax.ShapeDtypeStruct((K // 2, 2 * D), jnp.float32),
        scratch_types=[
            pltpu.VMEM((K,), jnp.int32),
            pltpu.VMEM((ROWS_PER_TILE,), jnp.int32),
            pltpu.VMEM((ROWS_PER_TILE,), jnp.int32),
            pltpu.VMEM((ROWS_PER_TILE, 2 * D), jnp.float32),
            pltpu.VMEM((ROWS_PER_TILE // 2, 2 * D), jnp.float32),
            pltpu.VMEM((ROWS_PER_TILE // 2, 2 * D), jnp.float32),
            pltpu.SemaphoreType.DMA,
        ],
    )(_gather_body)
    out2 = fn(x.reshape(N // 2, 2 * D), sorted_idx,
              scale_mat.reshape(K // 2, 2 * D))
    return out2.reshape(K, D)


def _select(ord1d):
    oi = lax.bitcast_convert_type(ord1d, jnp.int32) ^ jnp.int32(_MINT)
    top_oi, top_pos = lax.top_k(oi, K)
    win_ord = lax.bitcast_convert_type(top_oi ^ jnp.int32(_MINT), jnp.uint32)
    return win_ord, top_pos.astype(jnp.int32)


def kernel(x, p, k):
    ord1d = _scores(x, p)
    win_ord, win_idx = _select(ord1d)
    sorted_idx, scale_mat = _rank(win_ord, win_idx)
    return _gather_scale(x, sorted_idx.reshape(K), scale_mat)
